# Initial kernel scaffold; baseline (speedup 1.0000x reference)
import jax
import jax.numpy as jnp
from jax.experimental import pallas as pl


def kernel(p1, o1, p2, o2, p3, o3, p4, o4, p5, o5, x1, x2, x3, x4, x5_enc, x5_dec, dec5_l1_W, dec5_l1_b, dec5_bn1_g, dec5_bn1_b, dec5_l2_W, dec5_l2_b, dec5_bn2_g, dec5_bn2_b, dec4_l1_W, dec4_l1_b, dec4_bn1_g, dec4_bn1_b, dec4_l2_W, dec4_l2_b, dec4_bn2_g, dec4_bn2_b, dec3_l1_W, dec3_l1_b, dec3_bn1_g, dec3_bn1_b, dec3_l2_W, dec3_l2_b, dec3_bn2_g, dec3_bn2_b, dec2_l1_W, dec2_l1_b, dec2_bn1_g, dec2_bn1_b, dec2_l2_W, dec2_l2_b, dec2_bn2_g, dec2_bn2_b, dec1_l1_W, dec1_l1_b, dec1_bn1_g, dec1_bn1_b, dec1_l2_W, dec1_l2_b, dec1_bn2_g, dec1_bn2_b, c1_W, c1_b, c_bn_g, c_bn_b, c2_W, c2_b):
    raise NotImplementedError("write your pallas kernel here")


# speedup vs baseline: 4.4911x; 4.4911x over previous
import jax
import jax.numpy as jnp
from jax.experimental import pallas as pl
from jax.experimental.pallas import tpu as pltpu

_EPS = 1e-5
_HI = jax.lax.Precision.HIGHEST
_BIG = 1e30
_TILE = 400
_N1, _N2, _N3, _N4, _N5 = 10000, 2500, 625, 157, 40


def _bn(y, g, b):
    m = jnp.mean(y, axis=0)
    v = jnp.var(y, axis=0)
    return (y - m) / jnp.sqrt(v + _EPS) * g + b


def _mlp(x, WT, b, g, bb):
    y = jnp.dot(x, WT, precision=jax.lax.Precision.DEFAULT) + b
    return jax.nn.relu(_bn(y, g, bb))


def _d2(pf, pcT):
    dx = pf[:, 0:1] - pcT[0:1, :]
    dy = pf[:, 1:2] - pcT[1:2, :]
    dz = pf[:, 2:3] - pcT[2:3, :]
    return dx * dx + dy * dy + dz * dz


def _interp_matrix(d2):
    M, N = d2.shape
    ii = jax.lax.broadcasted_iota(jnp.int32, (M, N), 1)

    def min_argmin(d):
        m = jnp.min(d, axis=1, keepdims=True)
        i = jnp.min(jnp.where(d == m, ii, N), axis=1, keepdims=True)
        return m, i

    m1, i1 = min_argmin(d2)
    d2b = jnp.where(ii == i1, _BIG, d2)
    m2, i2 = min_argmin(d2b)
    d2c = jnp.where(ii == i2, _BIG, d2b)
    m3, i3 = min_argmin(d2c)

    w1 = 1.0 / (jnp.sqrt(jnp.maximum(m1, 1e-12)) + 1e-8)
    w2 = 1.0 / (jnp.sqrt(jnp.maximum(m2, 1e-12)) + 1e-8)
    w3 = 1.0 / (jnp.sqrt(jnp.maximum(m3, 1e-12)) + 1e-8)
    ws = w1 + w2 + w3
    C = (jnp.where(ii == i1, w1, 0.0)
         + jnp.where(ii == i2, w2, 0.0)
         + jnp.where(ii == i3, w3, 0.0))
    return C / ws


def _stage(a, f2, pf, pcT):
    C = _interp_matrix(_d2(pf, pcT))
    return a + jnp.dot(C, f2, precision=_HI)


def _body(p1_r, p2_r, p3_r, p4_r, p5_r,
          p2T_r, p3T_r, p4T_r, p5T_r,
          x1_r, x2_r, x3_r, x4_r, x5e_r, x5d_r,
          w51_r, b51_r, g51_r, c51_r, w52_r, b52_r, g52_r, c52_r,
          w41_r, b41_r, g41_r, c41_r, w42_r, b42_r, g42_r, c42_r,
          w31_r, b31_r, g31_r, c31_r, w32_r, b32_r, g32_r, c32_r,
          w21_r, b21_r, g21_r, c21_r, w22_r, b22_r, g22_r, c22_r,
          w11_r, b11_r, g11_r, c11_r, w12_r, b12_r, g12_r, c12_r,
          c1w_r, c1b_r, cg_r, cb_r, c2w_r, c2b_r,
          out_r, a1_scr, r1_scr):
    p5T = p5T_r[:, :]
    p4T = p4T_r[:, :]
    p3T = p3T_r[:, :]
    p2T = p2T_r[:, :]

    a5 = _mlp(x5d_r[:, :], w51_r[:, :], b51_r[:, :], g51_r[:, :], c51_r[:, :])
    f5 = _mlp(x5e_r[:, :], w52_r[:, :], b52_r[:, :], g52_r[:, :], c52_r[:, :])
    r5 = _stage(a5, f5, p5_r[:, :], p5T)

    a4 = _mlp(x4_r[:, :], w41_r[:, :], b41_r[:, :], g41_r[:, :], c41_r[:, :])
    f4 = _mlp(r5, w42_r[:, :], b42_r[:, :], g42_r[:, :], c42_r[:, :])
    r4 = _stage(a4, f4, p4_r[:, :], p5T)

    a3 = _mlp(x3_r[:, :], w31_r[:, :], b31_r[:, :], g31_r[:, :], c31_r[:, :])
    f3 = _mlp(r4, w32_r[:, :], b32_r[:, :], g32_r[:, :], c32_r[:, :])
    r3 = _stage(a3, f3, p3_r[:, :], p4T)

    a2 = _mlp(x2_r[:, :], w21_r[:, :], b21_r[:, :], g21_r[:, :], c21_r[:, :])
    f2 = _mlp(r3, w22_r[:, :], b22_r[:, :], g22_r[:, :], c22_r[:, :])
    r2 = _stage(a2, f2, p2_r[:, :], p3T)

    a1_scr[:, :] = _mlp(x1_r[:, :], w11_r[:, :], b11_r[:, :], g11_r[:, :],
                        c11_r[:, :])
    f1 = _mlp(r2, w12_r[:, :], b12_r[:, :], g12_r[:, :], c12_r[:, :])

    def tile_body(j, carry):
        rows = pl.ds(j * _TILE, _TILE)
        pf = p1_r[rows, :]
        C = _interp_matrix(_d2(pf, p2T))
        r1_scr[rows, :] = a1_scr[rows, :] + jnp.dot(C, f1, precision=_HI)
        return carry

    jax.lax.fori_loop(0, _N1 // _TILE, tile_body, 0)

    r1 = r1_scr[:, :]
    h = _mlp(r1, c1w_r[:, :], c1b_r[:, :], cg_r[:, :], cb_r[:, :])
    out_r[:, :] = jnp.dot(h, c2w_r[:, :],
                          precision=jax.lax.Precision.DEFAULT) + c2b_r[:, :]


def kernel(p1, o1, p2, o2, p3, o3, p4, o4, p5, o5,
           x1, x2, x3, x4, x5_enc, x5_dec,
           dec5_l1_W, dec5_l1_b, dec5_bn1_g, dec5_bn1_b,
           dec5_l2_W, dec5_l2_b, dec5_bn2_g, dec5_bn2_b,
           dec4_l1_W, dec4_l1_b, dec4_bn1_g, dec4_bn1_b,
           dec4_l2_W, dec4_l2_b, dec4_bn2_g, dec4_bn2_b,
           dec3_l1_W, dec3_l1_b, dec3_bn1_g, dec3_bn1_b,
           dec3_l2_W, dec3_l2_b, dec3_bn2_g, dec3_bn2_b,
           dec2_l1_W, dec2_l1_b, dec2_bn1_g, dec2_bn1_b,
           dec2_l2_W, dec2_l2_b, dec2_bn2_g, dec2_bn2_b,
           dec1_l1_W, dec1_l1_b, dec1_bn1_g, dec1_bn1_b,
           dec1_l2_W, dec1_l2_b, dec1_bn2_g, dec1_bn2_b,
           c1_W, c1_b, c_bn_g, c_bn_b, c2_W, c2_b):
    row = lambda v: v.reshape(1, -1)
    args = [
        p1, p2, p3, p4, p5,
        p2.T, p3.T, p4.T, p5.T,
        x1, x2, x3, x4, x5_enc, x5_dec,
        dec5_l1_W.T, row(dec5_l1_b), row(dec5_bn1_g), row(dec5_bn1_b),
        dec5_l2_W.T, row(dec5_l2_b), row(dec5_bn2_g), row(dec5_bn2_b),
        dec4_l1_W.T, row(dec4_l1_b), row(dec4_bn1_g), row(dec4_bn1_b),
        dec4_l2_W.T, row(dec4_l2_b), row(dec4_bn2_g), row(dec4_bn2_b),
        dec3_l1_W.T, row(dec3_l1_b), row(dec3_bn1_g), row(dec3_bn1_b),
        dec3_l2_W.T, row(dec3_l2_b), row(dec3_bn2_g), row(dec3_bn2_b),
        dec2_l1_W.T, row(dec2_l1_b), row(dec2_bn1_g), row(dec2_bn1_b),
        dec2_l2_W.T, row(dec2_l2_b), row(dec2_bn2_g), row(dec2_bn2_b),
        dec1_l1_W.T, row(dec1_l1_b), row(dec1_bn1_g), row(dec1_bn1_b),
        dec1_l2_W.T, row(dec1_l2_b), row(dec1_bn2_g), row(dec1_bn2_b),
        c1_W.T, row(c1_b), row(c_bn_g), row(c_bn_b),
        c2_W.T, row(c2_b),
    ]
    out = pl.pallas_call(
        _body,
        out_shape=jax.ShapeDtypeStruct((_N1, 1), jnp.float32),
        scratch_shapes=[
            pltpu.VMEM((_N1, 32), jnp.float32),
            pltpu.VMEM((_N1, 32), jnp.float32),
        ],
    )(*args)
    return out


# speedup vs baseline: 4.5956x; 1.0233x over previous
import jax
import jax.numpy as jnp
from jax.experimental import pallas as pl
from jax.experimental.pallas import tpu as pltpu

_EPS = 1e-5
_HI = jax.lax.Precision.HIGHEST
_BIG = 1e30
_TILE = 400
_N1, _N2, _N3, _N4, _N5 = 10000, 2500, 625, 157, 40


def _bn(y, g, b):
    m = jnp.mean(y, axis=0)
    v = jnp.var(y, axis=0)
    return (y - m) / jnp.sqrt(v + _EPS) * g + b


def _mlp(x, WT, b, g, bb):
    y = jnp.dot(x, WT, precision=jax.lax.Precision.DEFAULT) + b
    return jax.nn.relu(_bn(y, g, bb))


def _d2(pf, pcT):
    dx = pf[:, 0:1] - pcT[0:1, :]
    dy = pf[:, 1:2] - pcT[1:2, :]
    dz = pf[:, 2:3] - pcT[2:3, :]
    return dx * dx + dy * dy + dz * dz


def _interp_matrix(d2):
    M, N = d2.shape
    ii = jax.lax.broadcasted_iota(jnp.int32, (M, N), 1)

    def min_argmin(d):
        m = jnp.min(d, axis=1, keepdims=True)
        i = jnp.min(jnp.where(d == m, ii, N), axis=1, keepdims=True)
        return m, i

    m1, i1 = min_argmin(d2)
    e1 = ii == i1
    d2b = jnp.where(e1, _BIG, d2)
    m2, i2 = min_argmin(d2b)
    e2 = ii == i2
    d2c = jnp.where(e2, _BIG, d2b)
    m3, i3 = min_argmin(d2c)

    w1 = 1.0 / (jnp.sqrt(jnp.maximum(m1, 1e-12)) + 1e-8)
    w2 = 1.0 / (jnp.sqrt(jnp.maximum(m2, 1e-12)) + 1e-8)
    w3 = 1.0 / (jnp.sqrt(jnp.maximum(m3, 1e-12)) + 1e-8)
    ws = w1 + w2 + w3
    return jnp.where(e1, w1 / ws,
                     jnp.where(e2, w2 / ws,
                               jnp.where(ii == i3, w3 / ws, 0.0)))


def _stage(a, f2, pf, pcT):
    C = _interp_matrix(_d2(pf, pcT))
    return a + jnp.dot(C, f2, precision=_HI)


def _body(p1_r, p2_r, p3_r, p4_r, p5_r,
          p2T_r, p3T_r, p4T_r, p5T_r,
          x1_r, x2_r, x3_r, x4_r, x5e_r, x5d_r,
          w51_r, b51_r, g51_r, c51_r, w52_r, b52_r, g52_r, c52_r,
          w41_r, b41_r, g41_r, c41_r, w42_r, b42_r, g42_r, c42_r,
          w31_r, b31_r, g31_r, c31_r, w32_r, b32_r, g32_r, c32_r,
          w21_r, b21_r, g21_r, c21_r, w22_r, b22_r, g22_r, c22_r,
          w11_r, b11_r, g11_r, c11_r, w12_r, b12_r, g12_r, c12_r,
          c1w_r, c1b_r, cg_r, cb_r, c2w_r, c2b_r,
          out_r, a1_scr, r1_scr):
    p5T = p5T_r[:, :]
    p4T = p4T_r[:, :]
    p3T = p3T_r[:, :]
    p2T = p2T_r[:, :]

    a5 = _mlp(x5d_r[:, :], w51_r[:, :], b51_r[:, :], g51_r[:, :], c51_r[:, :])
    f5 = _mlp(x5e_r[:, :], w52_r[:, :], b52_r[:, :], g52_r[:, :], c52_r[:, :])
    r5 = _stage(a5, f5, p5_r[:, :], p5T)

    a4 = _mlp(x4_r[:, :], w41_r[:, :], b41_r[:, :], g41_r[:, :], c41_r[:, :])
    f4 = _mlp(r5, w42_r[:, :], b42_r[:, :], g42_r[:, :], c42_r[:, :])
    r4 = _stage(a4, f4, p4_r[:, :], p5T)

    a3 = _mlp(x3_r[:, :], w31_r[:, :], b31_r[:, :], g31_r[:, :], c31_r[:, :])
    f3 = _mlp(r4, w32_r[:, :], b32_r[:, :], g32_r[:, :], c32_r[:, :])
    r3 = _stage(a3, f3, p3_r[:, :], p4T)

    a2 = _mlp(x2_r[:, :], w21_r[:, :], b21_r[:, :], g21_r[:, :], c21_r[:, :])
    f2 = _mlp(r3, w22_r[:, :], b22_r[:, :], g22_r[:, :], c22_r[:, :])
    r2 = _stage(a2, f2, p2_r[:, :], p3T)

    a1_scr[:, :] = _mlp(x1_r[:, :], w11_r[:, :], b11_r[:, :], g11_r[:, :],
                        c11_r[:, :])
    f1 = _mlp(r2, w12_r[:, :], b12_r[:, :], g12_r[:, :], c12_r[:, :])

    def tile_body(j, carry):
        rows = pl.ds(j * _TILE, _TILE)
        pf = p1_r[rows, :]
        C = _interp_matrix(_d2(pf, p2T))
        r1_scr[rows, :] = a1_scr[rows, :] + jnp.dot(C, f1, precision=_HI)
        return carry

    jax.lax.fori_loop(0, _N1 // _TILE, tile_body, 0)

    r1 = r1_scr[:, :]
    h = _mlp(r1, c1w_r[:, :], c1b_r[:, :], cg_r[:, :], cb_r[:, :])
    out_r[:, :] = jnp.dot(h, c2w_r[:, :],
                          precision=jax.lax.Precision.DEFAULT) + c2b_r[:, :]


def kernel(p1, o1, p2, o2, p3, o3, p4, o4, p5, o5,
           x1, x2, x3, x4, x5_enc, x5_dec,
           dec5_l1_W, dec5_l1_b, dec5_bn1_g, dec5_bn1_b,
           dec5_l2_W, dec5_l2_b, dec5_bn2_g, dec5_bn2_b,
           dec4_l1_W, dec4_l1_b, dec4_bn1_g, dec4_bn1_b,
           dec4_l2_W, dec4_l2_b, dec4_bn2_g, dec4_bn2_b,
           dec3_l1_W, dec3_l1_b, dec3_bn1_g, dec3_bn1_b,
           dec3_l2_W, dec3_l2_b, dec3_bn2_g, dec3_bn2_b,
           dec2_l1_W, dec2_l1_b, dec2_bn1_g, dec2_bn1_b,
           dec2_l2_W, dec2_l2_b, dec2_bn2_g, dec2_bn2_b,
           dec1_l1_W, dec1_l1_b, dec1_bn1_g, dec1_bn1_b,
           dec1_l2_W, dec1_l2_b, dec1_bn2_g, dec1_bn2_b,
           c1_W, c1_b, c_bn_g, c_bn_b, c2_W, c2_b):
    row = lambda v: v.reshape(1, -1)
    args = [
        p1, p2, p3, p4, p5,
        p2.T, p3.T, p4.T, p5.T,
        x1, x2, x3, x4, x5_enc, x5_dec,
        dec5_l1_W.T, row(dec5_l1_b), row(dec5_bn1_g), row(dec5_bn1_b),
        dec5_l2_W.T, row(dec5_l2_b), row(dec5_bn2_g), row(dec5_bn2_b),
        dec4_l1_W.T, row(dec4_l1_b), row(dec4_bn1_g), row(dec4_bn1_b),
        dec4_l2_W.T, row(dec4_l2_b), row(dec4_bn2_g), row(dec4_bn2_b),
        dec3_l1_W.T, row(dec3_l1_b), row(dec3_bn1_g), row(dec3_bn1_b),
        dec3_l2_W.T, row(dec3_l2_b), row(dec3_bn2_g), row(dec3_bn2_b),
        dec2_l1_W.T, row(dec2_l1_b), row(dec2_bn1_g), row(dec2_bn1_b),
        dec2_l2_W.T, row(dec2_l2_b), row(dec2_bn2_g), row(dec2_bn2_b),
        dec1_l1_W.T, row(dec1_l1_b), row(dec1_bn1_g), row(dec1_bn1_b),
        dec1_l2_W.T, row(dec1_l2_b), row(dec1_bn2_g), row(dec1_bn2_b),
        c1_W.T, row(c1_b), row(c_bn_g), row(c_bn_b),
        c2_W.T, row(c2_b),
    ]
    out = pl.pallas_call(
        _body,
        out_shape=jax.ShapeDtypeStruct((_N1, 1), jnp.float32),
        scratch_shapes=[
            pltpu.VMEM((_N1, 32), jnp.float32),
            pltpu.VMEM((_N1, 32), jnp.float32),
        ],
    )(*args)
    return out


# speedup vs baseline: 5.3106x; 1.1556x over previous
import jax
import jax.numpy as jnp
from jax.experimental import pallas as pl
from jax.experimental.pallas import tpu as pltpu

_EPS = 1e-5
_HI = jax.lax.Precision.HIGHEST
_BIG = 1e30
_TILE = 400
_N1, _N2, _N3, _N4, _N5 = 10000, 2500, 625, 157, 40


def _bn(y, g, b):
    m = jnp.mean(y, axis=0)
    v = jnp.var(y, axis=0)
    return (y - m) / jnp.sqrt(v + _EPS) * g + b


def _mlp(x, WT, b, g, bb):
    y = jnp.dot(x, WT, precision=jax.lax.Precision.DEFAULT) + b
    return jax.nn.relu(_bn(y, g, bb))


def _d2(pf, pcT):
    dx = pf[:, 0:1] - pcT[0:1, :]
    dy = pf[:, 1:2] - pcT[1:2, :]
    dz = pf[:, 2:3] - pcT[2:3, :]
    return dx * dx + dy * dy + dz * dz


def _interp_matrix(d2):
    M, N = d2.shape
    ii = jax.lax.broadcasted_iota(jnp.int32, (M, N), 1)

    def min_argmin(d):
        m = jnp.min(d, axis=1, keepdims=True)
        i = jnp.min(jnp.where(d == m, ii, N), axis=1, keepdims=True)
        return m, i

    m1, i1 = min_argmin(d2)
    e1 = ii == i1
    d2b = jnp.where(e1, _BIG, d2)
    m2, i2 = min_argmin(d2b)
    e2 = ii == i2
    d2c = jnp.where(e2, _BIG, d2b)
    m3, i3 = min_argmin(d2c)

    w1 = 1.0 / (jnp.sqrt(jnp.maximum(m1, 1e-12)) + 1e-8)
    w2 = 1.0 / (jnp.sqrt(jnp.maximum(m2, 1e-12)) + 1e-8)
    w3 = 1.0 / (jnp.sqrt(jnp.maximum(m3, 1e-12)) + 1e-8)
    ws = w1 + w2 + w3
    return jnp.where(e1, w1 / ws,
                     jnp.where(e2, w2 / ws,
                               jnp.where(ii == i3, w3 / ws, 0.0)))


def _dot3(A, B):
    Ah = A.astype(jnp.bfloat16)
    Al = (A - Ah.astype(jnp.float32)).astype(jnp.bfloat16)
    Bh = B.astype(jnp.bfloat16)
    Bl = (B - Bh.astype(jnp.float32)).astype(jnp.bfloat16)
    d = lambda x, y: jnp.dot(x, y, preferred_element_type=jnp.float32)
    return d(Ah, Bh) + d(Ah, Bl) + d(Al, Bh)


def _stage(a, f2, pf, pcT):
    C = _interp_matrix(_d2(pf, pcT))
    return a + _dot3(C, f2)


def _body(p1_r, p2_r, p3_r, p4_r, p5_r,
          p2T_r, p3T_r, p4T_r, p5T_r,
          x1_r, x2_r, x3_r, x4_r, x5e_r, x5d_r,
          w51_r, b51_r, g51_r, c51_r, w52_r, b52_r, g52_r, c52_r,
          w41_r, b41_r, g41_r, c41_r, w42_r, b42_r, g42_r, c42_r,
          w31_r, b31_r, g31_r, c31_r, w32_r, b32_r, g32_r, c32_r,
          w21_r, b21_r, g21_r, c21_r, w22_r, b22_r, g22_r, c22_r,
          w11_r, b11_r, g11_r, c11_r, w12_r, b12_r, g12_r, c12_r,
          c1w_r, c1b_r, cg_r, cb_r, c2w_r, c2b_r,
          out_r, a1_scr, r1_scr):
    p5T = p5T_r[:, :]
    p4T = p4T_r[:, :]
    p3T = p3T_r[:, :]
    p2T = p2T_r[:, :]

    a5 = _mlp(x5d_r[:, :], w51_r[:, :], b51_r[:, :], g51_r[:, :], c51_r[:, :])
    f5 = _mlp(x5e_r[:, :], w52_r[:, :], b52_r[:, :], g52_r[:, :], c52_r[:, :])
    r5 = _stage(a5, f5, p5_r[:, :], p5T)

    a4 = _mlp(x4_r[:, :], w41_r[:, :], b41_r[:, :], g41_r[:, :], c41_r[:, :])
    f4 = _mlp(r5, w42_r[:, :], b42_r[:, :], g42_r[:, :], c42_r[:, :])
    r4 = _stage(a4, f4, p4_r[:, :], p5T)

    a3 = _mlp(x3_r[:, :], w31_r[:, :], b31_r[:, :], g31_r[:, :], c31_r[:, :])
    f3 = _mlp(r4, w32_r[:, :], b32_r[:, :], g32_r[:, :], c32_r[:, :])
    r3 = _stage(a3, f3, p3_r[:, :], p4T)

    a2 = _mlp(x2_r[:, :], w21_r[:, :], b21_r[:, :], g21_r[:, :], c21_r[:, :])
    f2 = _mlp(r3, w22_r[:, :], b22_r[:, :], g22_r[:, :], c22_r[:, :])
    r2 = _stage(a2, f2, p2_r[:, :], p3T)

    a1_scr[:, :] = _mlp(x1_r[:, :], w11_r[:, :], b11_r[:, :], g11_r[:, :],
                        c11_r[:, :])
    f1 = _mlp(r2, w12_r[:, :], b12_r[:, :], g12_r[:, :], c12_r[:, :])

    def tile_body(j, carry):
        rows = pl.ds(j * _TILE, _TILE)
        pf = p1_r[rows, :]
        C = _interp_matrix(_d2(pf, p2T))
        r1_scr[rows, :] = a1_scr[rows, :] + _dot3(C, f1)
        return carry

    jax.lax.fori_loop(0, _N1 // _TILE, tile_body, 0)

    r1 = r1_scr[:, :]
    h = _mlp(r1, c1w_r[:, :], c1b_r[:, :], cg_r[:, :], cb_r[:, :])
    out_r[:, :] = jnp.dot(h, c2w_r[:, :],
                          precision=jax.lax.Precision.DEFAULT) + c2b_r[:, :]


def kernel(p1, o1, p2, o2, p3, o3, p4, o4, p5, o5,
           x1, x2, x3, x4, x5_enc, x5_dec,
           dec5_l1_W, dec5_l1_b, dec5_bn1_g, dec5_bn1_b,
           dec5_l2_W, dec5_l2_b, dec5_bn2_g, dec5_bn2_b,
           dec4_l1_W, dec4_l1_b, dec4_bn1_g, dec4_bn1_b,
           dec4_l2_W, dec4_l2_b, dec4_bn2_g, dec4_bn2_b,
           dec3_l1_W, dec3_l1_b, dec3_bn1_g, dec3_bn1_b,
           dec3_l2_W, dec3_l2_b, dec3_bn2_g, dec3_bn2_b,
           dec2_l1_W, dec2_l1_b, dec2_bn1_g, dec2_bn1_b,
           dec2_l2_W, dec2_l2_b, dec2_bn2_g, dec2_bn2_b,
           dec1_l1_W, dec1_l1_b, dec1_bn1_g, dec1_bn1_b,
           dec1_l2_W, dec1_l2_b, dec1_bn2_g, dec1_bn2_b,
           c1_W, c1_b, c_bn_g, c_bn_b, c2_W, c2_b):
    row = lambda v: v.reshape(1, -1)
    args = [
        p1, p2, p3, p4, p5,
        p2.T, p3.T, p4.T, p5.T,
        x1, x2, x3, x4, x5_enc, x5_dec,
        dec5_l1_W.T, row(dec5_l1_b), row(dec5_bn1_g), row(dec5_bn1_b),
        dec5_l2_W.T, row(dec5_l2_b), row(dec5_bn2_g), row(dec5_bn2_b),
        dec4_l1_W.T, row(dec4_l1_b), row(dec4_bn1_g), row(dec4_bn1_b),
        dec4_l2_W.T, row(dec4_l2_b), row(dec4_bn2_g), row(dec4_bn2_b),
        dec3_l1_W.T, row(dec3_l1_b), row(dec3_bn1_g), row(dec3_bn1_b),
        dec3_l2_W.T, row(dec3_l2_b), row(dec3_bn2_g), row(dec3_bn2_b),
        dec2_l1_W.T, row(dec2_l1_b), row(dec2_bn1_g), row(dec2_bn1_b),
        dec2_l2_W.T, row(dec2_l2_b), row(dec2_bn2_g), row(dec2_bn2_b),
        dec1_l1_W.T, row(dec1_l1_b), row(dec1_bn1_g), row(dec1_bn1_b),
        dec1_l2_W.T, row(dec1_l2_b), row(dec1_bn2_g), row(dec1_bn2_b),
        c1_W.T, row(c1_b), row(c_bn_g), row(c_bn_b),
        c2_W.T, row(c2_b),
    ]
    out = pl.pallas_call(
        _body,
        out_shape=jax.ShapeDtypeStruct((_N1, 1), jnp.float32),
        scratch_shapes=[
            pltpu.VMEM((_N1, 32), jnp.float32),
            pltpu.VMEM((_N1, 32), jnp.float32),
        ],
    )(*args)
    return out


# speedup vs baseline: 6.2243x; 1.1721x over previous
import jax
import jax.numpy as jnp
from jax.experimental import pallas as pl
from jax.experimental.pallas import tpu as pltpu

_EPS = 1e-5
_HI = jax.lax.Precision.HIGHEST
_BIG = 1e30
_TILE = 400
_N1, _N2, _N3, _N4, _N5 = 10000, 2500, 625, 157, 40


def _bn(y, g, b):
    m = jnp.mean(y, axis=0)
    v = jnp.var(y, axis=0)
    return (y - m) / jnp.sqrt(v + _EPS) * g + b


def _mlp(x, WT, b, g, bb):
    y = jnp.dot(x, WT, precision=jax.lax.Precision.DEFAULT) + b
    return jax.nn.relu(_bn(y, g, bb))


def _d2(pf, pcT):
    dx = pf[:, 0:1] - pcT[0:1, :]
    dy = pf[:, 1:2] - pcT[1:2, :]
    dz = pf[:, 2:3] - pcT[2:3, :]
    return dx * dx + dy * dy + dz * dz


def _top3_values(d2, chunk=128):
    M, N = d2.shape
    m1 = jnp.full((M, chunk), _BIG, jnp.float32)
    m2 = m1
    m3 = m1
    for k in range(0, N, chunk):
        v = d2[:, k:k + chunk]
        t2 = jnp.maximum(m1, v)
        m1 = jnp.minimum(m1, v)
        t3 = jnp.maximum(m2, t2)
        m2 = jnp.minimum(m2, t2)
        m3 = jnp.minimum(m3, t3)
    U = jnp.concatenate([m1, m2, m3], axis=1)
    K = 3 * chunk
    jj = jax.lax.broadcasted_iota(jnp.int32, (M, K), 1)

    def min_argmin(u):
        g = jnp.min(u, axis=1, keepdims=True)
        i = jnp.min(jnp.where(u == g, jj, K), axis=1, keepdims=True)
        return g, i

    g1, j1 = min_argmin(U)
    Ub = jnp.where(jj == j1, _BIG, U)
    g2, j2 = min_argmin(Ub)
    Uc = jnp.where(jj == j2, _BIG, Ub)
    g3 = jnp.min(Uc, axis=1, keepdims=True)
    return g1, g2, g3


def _interp_matrix(d2):
    g1, g2, g3 = _top3_values(d2)
    w1 = 1.0 / (jnp.sqrt(jnp.maximum(g1, 1e-12)) + 1e-8)
    w2 = 1.0 / (jnp.sqrt(jnp.maximum(g2, 1e-12)) + 1e-8)
    w3 = 1.0 / (jnp.sqrt(jnp.maximum(g3, 1e-12)) + 1e-8)
    ws = w1 + w2 + w3
    return jnp.where(d2 == g1, w1 / ws,
                     jnp.where(d2 == g2, w2 / ws,
                               jnp.where(d2 == g3, w3 / ws, 0.0)))


def _dot3(A, B):
    Ah = A.astype(jnp.bfloat16)
    Al = (A - Ah.astype(jnp.float32)).astype(jnp.bfloat16)
    Bh = B.astype(jnp.bfloat16)
    Bl = (B - Bh.astype(jnp.float32)).astype(jnp.bfloat16)
    d = lambda x, y: jnp.dot(x, y, preferred_element_type=jnp.float32)
    return d(Ah, Bh) + d(Ah, Bl) + d(Al, Bh)


def _pad_rows(f2, npad):
    n, c = f2.shape
    if n == npad:
        return f2
    return jnp.concatenate([f2, jnp.zeros((npad - n, c), jnp.float32)], axis=0)


def _stage(a, f2, pf, pcT):
    C = _interp_matrix(_d2(pf, pcT))
    return a + _dot3(C, _pad_rows(f2, pcT.shape[1]))


def _body(p1_r, p2_r, p3_r, p4_r, p5_r,
          p2T_r, p3T_r, p4T_r, p5T_r,
          x1_r, x2_r, x3_r, x4_r, x5e_r, x5d_r,
          w51_r, b51_r, g51_r, c51_r, w52_r, b52_r, g52_r, c52_r,
          w41_r, b41_r, g41_r, c41_r, w42_r, b42_r, g42_r, c42_r,
          w31_r, b31_r, g31_r, c31_r, w32_r, b32_r, g32_r, c32_r,
          w21_r, b21_r, g21_r, c21_r, w22_r, b22_r, g22_r, c22_r,
          w11_r, b11_r, g11_r, c11_r, w12_r, b12_r, g12_r, c12_r,
          c1w_r, c1b_r, cg_r, cb_r, c2w_r, c2b_r,
          out_r, a1_scr, r1_scr):
    p5T = p5T_r[:, :]
    p4T = p4T_r[:, :]
    p3T = p3T_r[:, :]
    p2T = p2T_r[:, :]

    a5 = _mlp(x5d_r[:, :], w51_r[:, :], b51_r[:, :], g51_r[:, :], c51_r[:, :])
    f5 = _mlp(x5e_r[:, :], w52_r[:, :], b52_r[:, :], g52_r[:, :], c52_r[:, :])
    r5 = _stage(a5, f5, p5_r[:, :], p5T)

    a4 = _mlp(x4_r[:, :], w41_r[:, :], b41_r[:, :], g41_r[:, :], c41_r[:, :])
    f4 = _mlp(r5, w42_r[:, :], b42_r[:, :], g42_r[:, :], c42_r[:, :])
    r4 = _stage(a4, f4, p4_r[:, :], p5T)

    a3 = _mlp(x3_r[:, :], w31_r[:, :], b31_r[:, :], g31_r[:, :], c31_r[:, :])
    f3 = _mlp(r4, w32_r[:, :], b32_r[:, :], g32_r[:, :], c32_r[:, :])
    r3 = _stage(a3, f3, p3_r[:, :], p4T)

    a2 = _mlp(x2_r[:, :], w21_r[:, :], b21_r[:, :], g21_r[:, :], c21_r[:, :])
    f2 = _mlp(r3, w22_r[:, :], b22_r[:, :], g22_r[:, :], c22_r[:, :])
    r2 = _stage(a2, f2, p2_r[:, :], p3T)

    a1_scr[:, :] = _mlp(x1_r[:, :], w11_r[:, :], b11_r[:, :], g11_r[:, :],
                        c11_r[:, :])
    f1 = _pad_rows(_mlp(r2, w12_r[:, :], b12_r[:, :], g12_r[:, :],
                        c12_r[:, :]), p2T.shape[1])

    def tile_body(j, carry):
        rows = pl.ds(j * _TILE, _TILE)
        pf = p1_r[rows, :]
        C = _interp_matrix(_d2(pf, p2T))
        r1_scr[rows, :] = a1_scr[rows, :] + _dot3(C, f1)
        return carry

    jax.lax.fori_loop(0, _N1 // _TILE, tile_body, 0)

    r1 = r1_scr[:, :]
    h = _mlp(r1, c1w_r[:, :], c1b_r[:, :], cg_r[:, :], cb_r[:, :])
    out_r[:, :] = jnp.dot(h, c2w_r[:, :],
                          precision=jax.lax.Precision.DEFAULT) + c2b_r[:, :]


def kernel(p1, o1, p2, o2, p3, o3, p4, o4, p5, o5,
           x1, x2, x3, x4, x5_enc, x5_dec,
           dec5_l1_W, dec5_l1_b, dec5_bn1_g, dec5_bn1_b,
           dec5_l2_W, dec5_l2_b, dec5_bn2_g, dec5_bn2_b,
           dec4_l1_W, dec4_l1_b, dec4_bn1_g, dec4_bn1_b,
           dec4_l2_W, dec4_l2_b, dec4_bn2_g, dec4_bn2_b,
           dec3_l1_W, dec3_l1_b, dec3_bn1_g, dec3_bn1_b,
           dec3_l2_W, dec3_l2_b, dec3_bn2_g, dec3_bn2_b,
           dec2_l1_W, dec2_l1_b, dec2_bn1_g, dec2_bn1_b,
           dec2_l2_W, dec2_l2_b, dec2_bn2_g, dec2_bn2_b,
           dec1_l1_W, dec1_l1_b, dec1_bn1_g, dec1_bn1_b,
           dec1_l2_W, dec1_l2_b, dec1_bn2_g, dec1_bn2_b,
           c1_W, c1_b, c_bn_g, c_bn_b, c2_W, c2_b):
    row = lambda v: v.reshape(1, -1)

    def padT(p):
        n = p.shape[0]
        npad = -(-n // 128) * 128
        return jnp.concatenate(
            [p.T, jnp.full((3, npad - n), 1e4, jnp.float32)], axis=1)

    args = [
        p1, p2, p3, p4, p5,
        padT(p2), padT(p3), padT(p4), padT(p5),
        x1, x2, x3, x4, x5_enc, x5_dec,
        dec5_l1_W.T, row(dec5_l1_b), row(dec5_bn1_g), row(dec5_bn1_b),
        dec5_l2_W.T, row(dec5_l2_b), row(dec5_bn2_g), row(dec5_bn2_b),
        dec4_l1_W.T, row(dec4_l1_b), row(dec4_bn1_g), row(dec4_bn1_b),
        dec4_l2_W.T, row(dec4_l2_b), row(dec4_bn2_g), row(dec4_bn2_b),
        dec3_l1_W.T, row(dec3_l1_b), row(dec3_bn1_g), row(dec3_bn1_b),
        dec3_l2_W.T, row(dec3_l2_b), row(dec3_bn2_g), row(dec3_bn2_b),
        dec2_l1_W.T, row(dec2_l1_b), row(dec2_bn1_g), row(dec2_bn1_b),
        dec2_l2_W.T, row(dec2_l2_b), row(dec2_bn2_g), row(dec2_bn2_b),
        dec1_l1_W.T, row(dec1_l1_b), row(dec1_bn1_g), row(dec1_bn1_b),
        dec1_l2_W.T, row(dec1_l2_b), row(dec1_bn2_g), row(dec1_bn2_b),
        c1_W.T, row(c1_b), row(c_bn_g), row(c_bn_b),
        c2_W.T, row(c2_b),
    ]
    out = pl.pallas_call(
        _body,
        out_shape=jax.ShapeDtypeStruct((_N1, 1), jnp.float32),
        scratch_shapes=[
            pltpu.VMEM((_N1, 32), jnp.float32),
            pltpu.VMEM((_N1, 32), jnp.float32),
        ],
    )(*args)
    return out
